# baseline (speedup 1.0000x reference)
import functools

import jax
import jax.numpy as jnp
from jax import lax
from jax.experimental import pallas as pl
from jax.experimental.pallas import tpu as pltpu

N = 10000
E = 320000
D = 64
T = 3
H = 4
NG = 16

_BLK = 2000


def _leaky(x):
    return jnp.where(x > 0, x, 0.2 * x)


def _gamma_body(xb_ref, xs_ref, xsum_ref, g0t, g1t, g2t, g3t, g4w, out_ref):
    a = jnp.dot(xb_ref[...], g0t[...], preferred_element_type=jnp.float32)
    b = jnp.dot(xs_ref[...], g1t[...], preferred_element_type=jnp.float32)
    c = jnp.dot(xsum_ref[...], g2t[...], preferred_element_type=jnp.float32)
    xc = jnp.concatenate([a, b, c], axis=-1)
    t = _leaky(jnp.dot(xc, g3t[...], preferred_element_type=jnp.float32))
    out_ref[...] = jnp.dot(t, g4w[...], preferred_element_type=jnp.float32)


def _gamma_stage(xb, x_s_rep, x_sum_rep, p):
    g0t = p['gamma0'].T
    g1t = p['gamma1'].T
    g2t = p['gamma2'].T
    g3t = p['gamma3'].T
    g4w = p['gamma4_W'].T
    row = lambda i: (i, 0)
    full = lambda shape: pl.BlockSpec(shape, lambda i: (0, 0))
    out = pl.pallas_call(
        _gamma_body,
        grid=(N // _BLK,),
        in_specs=[
            pl.BlockSpec((_BLK, D), row),
            pl.BlockSpec((_BLK, D), row),
            pl.BlockSpec((_BLK, D), row),
            full((D, D)), full((D, D)), full((D, D)),
            full((3 * D, 3 * D // 2)), full((3 * D // 2, 1)),
        ],
        out_specs=pl.BlockSpec((_BLK, 1), row),
        out_shape=jax.ShapeDtypeStruct((N, 1), jnp.float32),
    )(xb, x_s_rep, x_sum_rep, g0t, g1t, g2t, g3t, g4w)
    return out[:, 0] + p['gamma4_b'][0]


def _gcn(x, src, dst, ew, W, b, n):
    h = x @ W.T + b
    return jax.ops.segment_sum(ew[:, None] * h[src], dst, num_segments=n)


def _mha(x, params, pre):
    n, L, d = x.shape
    dh = d // H
    q = (x @ params[pre + '_wq'].T).reshape(n, L, H, dh).transpose(0, 2, 1, 3).reshape(n * H, L, dh)
    kk = (x @ params[pre + '_wk'].T).reshape(n, L, H, dh).transpose(0, 2, 1, 3).reshape(n * H, L, dh)
    v = (x @ params[pre + '_wv'].T).reshape(n, L, H, dh).transpose(0, 2, 1, 3).reshape(n * H, L, dh)
    scores = jnp.tanh(q[:, :, None, :] + kk[:, None, :, :]) @ params[pre + '_ws'].T
    aw = jax.nn.softmax(scores, axis=-1)[..., 0]
    nv = jnp.einsum('bij,bjd->bid', aw, v)
    out = nv.reshape(n, H, L, dh).transpose(0, 2, 1, 3).reshape(n, L, d)
    return out @ params[pre + '_wo'].T


def kernel(x, edge_index, edge_weight, batch, states, params):
    p = params
    n = x.shape[0]
    src = edge_index[0]
    dst = edge_index[1]
    nodes = jnp.arange(n, dtype=edge_index.dtype)
    ones_e = jnp.ones((n,), dtype=jnp.float32)
    x1 = x[:, D:]
    x2 = x[:, :D]
    for t in range(T):
        W = p['enc_cW%d' % t]
        b = p['enc_cb%d' % t]
        if t == 0:
            x1 = _leaky(_gcn(x1, dst, src, edge_weight, W, b, n))
            x2 = _leaky(_gcn(x2, src, dst, edge_weight, W, b, n))
        else:
            ew2 = jnp.concatenate([edge_weight, ones_e])
            x1 = _leaky(_gcn(x1, jnp.concatenate([dst, nodes]), jnp.concatenate([src, nodes]), ew2, W, b, n))
            x2 = _leaky(_gcn(x2, jnp.concatenate([src, nodes]), jnp.concatenate([dst, nodes]), ew2, W, b, n))
    y = jnp.concatenate([x1 @ p['enc_w1'].T, x2 @ p['enc_w2'].T], axis=-1) @ p['enc_fc1_W'].T + p['enc_fc1_b']
    x1 = y[:, :D]
    x2 = y[:, D:]
    x1_list = []
    x2_list = []
    for t in range(T):
        h1 = _leaky((x1 + states[:, None] * p['b1_%d_a0' % t][0, 0]) @ p['b1_%d_a1W' % t].T + p['b1_%d_a1b' % t])
        x1 = _leaky(_gcn(h1, dst, src, edge_weight, p['b1_%d_cW' % t], p['b1_%d_cb' % t], n))
        h2 = _leaky((x2 + states[:, None] * p['b2_%d_a0' % t][0, 0]) @ p['b2_%d_a1W' % t].T + p['b2_%d_a1b' % t])
        x2 = _leaky(_gcn(h2, src, dst, edge_weight, p['b2_%d_cW' % t], p['b2_%d_cb' % t], n))
        x1_list.append(x1)
        x2_list.append(x2)
    x1s = jnp.stack(x1_list, axis=1)
    x2s = jnp.stack(x2_list, axis=1)
    x1_a = _mha(x1s, p, 'att1')
    x2_a = _mha(x2s, p, 'att2')
    aw1 = jax.nn.softmax((x1_a @ p['fc1_W'].T + p['fc1_b']).transpose(0, 2, 1), axis=-1)
    aw2 = jax.nn.softmax((x2_a @ p['fc2_W'].T + p['fc2_b']).transpose(0, 2, 1), axis=-1)
    x1f = jnp.einsum('bij,bjd->bid', aw1, x1_a)[:, 0, :] + x1s.sum(axis=1)
    x2f = jnp.einsum('bij,bjd->bid', aw2, x2_a)[:, 0, :] + x2s.sum(axis=1)
    xb = _leaky(jnp.concatenate([x1f @ p['beta0'].T, x2f @ p['beta1'].T], axis=-1) @ p['beta2_W'].T + p['beta2_b'])
    sel = states == 1.0
    batch_num = jax.ops.segment_sum(jnp.ones((n,), dtype=jnp.int32), batch, num_segments=NG)
    x_s = jax.ops.segment_sum(jnp.where(sel[:, None], xb, 0.0), batch, num_segments=NG)
    x_sum = jax.ops.segment_sum(xb, batch, num_segments=NG)
    x_s_rep = jnp.repeat(x_s, batch_num, axis=0, total_repeat_length=n)
    x_sum_rep = jnp.repeat(x_sum, batch_num, axis=0, total_repeat_length=n)
    return _gamma_stage(xb, x_s_rep, x_sum_rep, p)


# speedup vs baseline: 3.7059x; 3.7059x over previous
import functools

import jax
import jax.numpy as jnp
from jax import lax
from jax.experimental import pallas as pl
from jax.experimental.pallas import tpu as pltpu
from jax.experimental.pallas import tpu_sc as plsc

N = 10000
E = 320000
D = 64
T = 3
H = 4
NG = 16

_BLK = 2000

_NC = 2
_NS = 16
_NW = _NC * _NS
_EPW = E // _NW
_C = 80
_NCH = _EPW // _C
_NP = 10240
_STRIPE = _NP // _NS


def _spmm2_body(h1_hbm, h2_hbm, src_hbm, dst_hbm, ew_hbm, init1_hbm, init2_hbm,
                out1_hbm, out2_hbm, si_v, di_v, ew_v, rows_v, acc1_sh, acc2_sh, sem):
    c = lax.axis_index("c")
    s = lax.axis_index("s")
    wid = c * _NS + s
    pltpu.sync_copy(src_hbm.at[wid], si_v)
    pltpu.sync_copy(dst_hbm.at[wid], di_v)
    pltpu.sync_copy(ew_hbm.at[wid], ew_v)
    pltpu.sync_copy(init1_hbm.at[c, pl.ds(s * _STRIPE, _STRIPE)],
                    acc1_sh.at[pl.ds(s * _STRIPE, _STRIPE)])
    pltpu.sync_copy(init2_hbm.at[c, pl.ds(s * _STRIPE, _STRIPE)],
                    acc2_sh.at[pl.ds(s * _STRIPE, _STRIPE)])
    plsc.subcore_barrier()

    def do_dir(h_hbm, acc_sh, g_v, sc_v):
        def chunk(i, carry):
            pltpu.async_copy(h_hbm.at[g_v.at[i]], rows_v, sem).wait()

            def row16(e16, carry2):
                wv = ew_v[i, pl.ds(e16 * 16, 16)]
                for k in range(16):
                    e = e16 * 16 + k
                    w = wv[k]
                    for j in range(D // 16):
                        rows_v[e, pl.ds(j * 16, 16)] = rows_v[e, pl.ds(j * 16, 16)] * w
                return carry2

            lax.fori_loop(0, _C // 16, row16, 0)
            pltpu.sync_copy(rows_v, acc_sh.at[sc_v.at[i]], add=True)
            return carry

        lax.fori_loop(0, _NCH, chunk, 0)

    do_dir(h1_hbm, acc1_sh, di_v, si_v)
    do_dir(h2_hbm, acc2_sh, si_v, di_v)
    plsc.subcore_barrier()
    pltpu.sync_copy(acc1_sh.at[pl.ds(s * _STRIPE, _STRIPE)],
                    out1_hbm.at[c, pl.ds(s * _STRIPE, _STRIPE)])
    pltpu.sync_copy(acc2_sh.at[pl.ds(s * _STRIPE, _STRIPE)],
                    out2_hbm.at[c, pl.ds(s * _STRIPE, _STRIPE)])


@jax.jit
def _spmm2(h1, h2, src3, dst3, ew3, init1, init2):
    mesh = plsc.VectorSubcoreMesh(core_axis_name="c", subcore_axis_name="s")
    f = pl.kernel(
        _spmm2_body,
        out_type=(jax.ShapeDtypeStruct((_NC, _NP, D), jnp.float32),
                  jax.ShapeDtypeStruct((_NC, _NP, D), jnp.float32)),
        mesh=mesh,
        scratch_types=[
            pltpu.VMEM((_NCH, _C), jnp.int32),
            pltpu.VMEM((_NCH, _C), jnp.int32),
            pltpu.VMEM((_NCH, _C), jnp.float32),
            pltpu.VMEM((_C, D), jnp.float32),
            pltpu.VMEM_SHARED((_NP, D), jnp.float32),
            pltpu.VMEM_SHARED((_NP, D), jnp.float32),
            pltpu.SemaphoreType.DMA,
        ],
        compiler_params=pltpu.CompilerParams(use_tc_tiling_on_sc=False),
    )
    return f(h1, h2, src3, dst3, ew3, init1, init2)


def _leaky(x):
    return jnp.where(x > 0, x, 0.2 * x)


def _gamma_body(xb_ref, xs_ref, xsum_ref, g0t, g1t, g2t, g3t, g4w, out_ref):
    a = jnp.dot(xb_ref[...], g0t[...], preferred_element_type=jnp.float32)
    b = jnp.dot(xs_ref[...], g1t[...], preferred_element_type=jnp.float32)
    c = jnp.dot(xsum_ref[...], g2t[...], preferred_element_type=jnp.float32)
    xc = jnp.concatenate([a, b, c], axis=-1)
    t = _leaky(jnp.dot(xc, g3t[...], preferred_element_type=jnp.float32))
    out_ref[...] = jnp.dot(t, g4w[...], preferred_element_type=jnp.float32)


def _gamma_stage(xb, x_s_rep, x_sum_rep, p):
    g0t = p['gamma0'].T
    g1t = p['gamma1'].T
    g2t = p['gamma2'].T
    g3t = p['gamma3'].T
    g4w = p['gamma4_W'].T
    row = lambda i: (i, 0)
    full = lambda shape: pl.BlockSpec(shape, lambda i: (0, 0))
    out = pl.pallas_call(
        _gamma_body,
        grid=(N // _BLK,),
        in_specs=[
            pl.BlockSpec((_BLK, D), row),
            pl.BlockSpec((_BLK, D), row),
            pl.BlockSpec((_BLK, D), row),
            full((D, D)), full((D, D)), full((D, D)),
            full((3 * D, 3 * D // 2)), full((3 * D // 2, 1)),
        ],
        out_specs=pl.BlockSpec((_BLK, 1), row),
        out_shape=jax.ShapeDtypeStruct((N, 1), jnp.float32),
    )(xb, x_s_rep, x_sum_rep, g0t, g1t, g2t, g3t, g4w)
    return out[:, 0] + p['gamma4_b'][0]


def _mha(x, params, pre):
    n, L, d = x.shape
    dh = d // H
    q = (x @ params[pre + '_wq'].T).reshape(n, L, H, dh).transpose(0, 2, 1, 3).reshape(n * H, L, dh)
    kk = (x @ params[pre + '_wk'].T).reshape(n, L, H, dh).transpose(0, 2, 1, 3).reshape(n * H, L, dh)
    v = (x @ params[pre + '_wv'].T).reshape(n, L, H, dh).transpose(0, 2, 1, 3).reshape(n * H, L, dh)
    scores = jnp.tanh(q[:, :, None, :] + kk[:, None, :, :]) @ params[pre + '_ws'].T
    aw = jax.nn.softmax(scores, axis=-1)[..., 0]
    nv = jnp.einsum('bij,bjd->bid', aw, v)
    out = nv.reshape(n, H, L, dh).transpose(0, 2, 1, 3).reshape(n, L, d)
    return out @ params[pre + '_wo'].T


def kernel(x, edge_index, edge_weight, batch, states, params):
    p = params
    n = x.shape[0]
    src3 = edge_index[0].reshape(_NW, _NCH, _C)
    dst3 = edge_index[1].reshape(_NW, _NCH, _C)
    ew3 = edge_weight.reshape(_NW, _NCH, _C)
    zeros2 = jnp.zeros((_NC, _NP, D), jnp.float32)
    padrows = jnp.zeros((_NP - N, D), jnp.float32)
    pad2 = lambda h: jnp.concatenate([h, padrows])[None]
    x1 = x[:, D:]
    x2 = x[:, :D]
    for t in range(T):
        W = p['enc_cW%d' % t]
        b = p['enc_cb%d' % t]
        h1 = x1 @ W.T + b
        h2 = x2 @ W.T + b
        if t == 0:
            init1 = init2 = zeros2
        else:
            init1 = jnp.concatenate([pad2(h1), zeros2[:1]], axis=0)
            init2 = jnp.concatenate([pad2(h2), zeros2[:1]], axis=0)
        o1, o2 = _spmm2(h1, h2, src3, dst3, ew3, init1, init2)
        x1 = _leaky(o1[0, :N] + o1[1, :N])
        x2 = _leaky(o2[0, :N] + o2[1, :N])
    y = jnp.concatenate([x1 @ p['enc_w1'].T, x2 @ p['enc_w2'].T], axis=-1) @ p['enc_fc1_W'].T + p['enc_fc1_b']
    x1 = y[:, :D]
    x2 = y[:, D:]
    x1_list = []
    x2_list = []
    for t in range(T):
        g1 = _leaky((x1 + states[:, None] * p['b1_%d_a0' % t][0, 0]) @ p['b1_%d_a1W' % t].T + p['b1_%d_a1b' % t])
        g2 = _leaky((x2 + states[:, None] * p['b2_%d_a0' % t][0, 0]) @ p['b2_%d_a1W' % t].T + p['b2_%d_a1b' % t])
        h1 = g1 @ p['b1_%d_cW' % t].T + p['b1_%d_cb' % t]
        h2 = g2 @ p['b2_%d_cW' % t].T + p['b2_%d_cb' % t]
        o1, o2 = _spmm2(h1, h2, src3, dst3, ew3, zeros2, zeros2)
        x1 = _leaky(o1[0, :N] + o1[1, :N])
        x2 = _leaky(o2[0, :N] + o2[1, :N])
        x1_list.append(x1)
        x2_list.append(x2)
    x1s = jnp.stack(x1_list, axis=1)
    x2s = jnp.stack(x2_list, axis=1)
    x1_a = _mha(x1s, p, 'att1')
    x2_a = _mha(x2s, p, 'att2')
    aw1 = jax.nn.softmax((x1_a @ p['fc1_W'].T + p['fc1_b']).transpose(0, 2, 1), axis=-1)
    aw2 = jax.nn.softmax((x2_a @ p['fc2_W'].T + p['fc2_b']).transpose(0, 2, 1), axis=-1)
    x1f = jnp.einsum('bij,bjd->bid', aw1, x1_a)[:, 0, :] + x1s.sum(axis=1)
    x2f = jnp.einsum('bij,bjd->bid', aw2, x2_a)[:, 0, :] + x2s.sum(axis=1)
    xb = _leaky(jnp.concatenate([x1f @ p['beta0'].T, x2f @ p['beta1'].T], axis=-1) @ p['beta2_W'].T + p['beta2_b'])
    sel = states == 1.0
    batch_num = jax.ops.segment_sum(jnp.ones((n,), dtype=jnp.int32), batch, num_segments=NG)
    x_s = jax.ops.segment_sum(jnp.where(sel[:, None], xb, 0.0), batch, num_segments=NG)
    x_sum = jax.ops.segment_sum(xb, batch, num_segments=NG)
    x_s_rep = jnp.repeat(x_s, batch_num, axis=0, total_repeat_length=n)
    x_sum_rep = jnp.repeat(x_sum, batch_num, axis=0, total_repeat_length=n)
    return _gamma_stage(xb, x_s_rep, x_sum_rep, p)


# speedup vs baseline: 5.7538x; 1.5526x over previous
import functools

import jax
import jax.numpy as jnp
from jax import lax
from jax.experimental import pallas as pl
from jax.experimental.pallas import tpu as pltpu
from jax.experimental.pallas import tpu_sc as plsc

N = 10000
E = 320000
D = 64
T = 3
H = 4
NG = 16

_BLK = 2000

_NC = 2
_NS = 16
_NW = _NC * _NS
_EPW = E // _NW
_C = 80
_NCH = _EPW // _C
_NP = 10240
_STRIPE = _NP // _NS


_NB = 5


def _spmm2_body(h1_hbm, h2_hbm, src_hbm, dst_hbm, ew_hbm, init1_hbm, init2_hbm,
                out1_hbm, out2_hbm, si_v, di_v, ew_v, rows_v, acc_sh,
                *sems):
    gsems = sems[:_NB]
    ssems = sems[_NB:]
    c = lax.axis_index("c")
    s = lax.axis_index("s")
    wid = c * _NS + s
    stripe = pl.ds(s * _STRIPE, _STRIPE)
    pltpu.sync_copy(src_hbm.at[wid], si_v)
    pltpu.sync_copy(dst_hbm.at[wid], di_v)
    pltpu.sync_copy(ew_hbm.at[wid], ew_v)
    pltpu.sync_copy(init1_hbm.at[c, stripe], acc_sh.at[stripe])
    plsc.subcore_barrier()

    def do_dir(h_hbm, acc_sh, g_v, sc_v):
        def gstart(i, b):
            pltpu.async_copy(h_hbm.at[g_v.at[i]], rows_v.at[b], gsems[b])

        def gwait(i, b):
            pltpu.make_async_copy(h_hbm.at[g_v.at[i]], rows_v.at[b], gsems[b]).wait()

        def sstart(i, b):
            pltpu.async_copy(rows_v.at[b], acc_sh.at[sc_v.at[i]], ssems[b], add=True)

        def swait(i, b):
            pltpu.make_async_copy(rows_v.at[b], acc_sh.at[sc_v.at[i]], ssems[b]).wait()

        def scale(i, b):
            def row16(e16, carry):
                wv = ew_v[i, pl.ds(e16 * 16, 16)]
                for k in range(16):
                    e = e16 * 16 + k
                    w = wv[k]
                    for j in range(D // 16):
                        rows_v[b, e, pl.ds(j * 16, 16)] = rows_v[b, e, pl.ds(j * 16, 16)] * w
                return carry

            lax.fori_loop(0, _C // 16, row16, 0)

        def slot(i, b, refill):
            gwait(i, b)
            scale(i, b)
            sstart(i, b)
            pb = (b - 1) % _NB
            if refill:
                swait(i - 1, pb)
                gstart(i - 1 + _NB, pb)

        for b in range(_NB):
            gstart(b, b)
        for b in range(_NB):
            slot(b, b, refill=(b >= 1))

        @pl.loop(1, _NCH // _NB - 1)
        def middle(g):
            i0 = g * _NB
            for b in range(_NB):
                slot(i0 + b, b, refill=True)

        i0 = _NCH - _NB
        for b in range(_NB):
            slot(i0 + b, b, refill=(b == 0))
            if b >= 1:
                swait(i0 + b - 1, (b - 1) % _NB)
        swait(_NCH - 1, (_NCH - 1) % _NB)

    do_dir(h1_hbm, acc_sh, di_v, si_v)
    plsc.subcore_barrier()
    pltpu.sync_copy(acc_sh.at[stripe], out1_hbm.at[c, stripe])
    pltpu.sync_copy(init2_hbm.at[c, stripe], acc_sh.at[stripe])
    plsc.subcore_barrier()
    do_dir(h2_hbm, acc_sh, si_v, di_v)
    plsc.subcore_barrier()
    pltpu.sync_copy(acc_sh.at[stripe], out2_hbm.at[c, stripe])


@jax.jit
def _spmm2(h1, h2, src3, dst3, ew3, init1, init2):
    mesh = plsc.VectorSubcoreMesh(core_axis_name="c", subcore_axis_name="s")
    f = pl.kernel(
        _spmm2_body,
        out_type=(jax.ShapeDtypeStruct((_NC, _NP, D), jnp.float32),
                  jax.ShapeDtypeStruct((_NC, _NP, D), jnp.float32)),
        mesh=mesh,
        scratch_types=[
            pltpu.VMEM((_NCH, _C), jnp.int32),
            pltpu.VMEM((_NCH, _C), jnp.int32),
            pltpu.VMEM((_NCH, _C), jnp.float32),
            pltpu.VMEM((_NB, _C, D), jnp.float32),
            pltpu.VMEM_SHARED((_NP, D), jnp.float32),
        ] + [pltpu.SemaphoreType.DMA] * (2 * _NB),
        compiler_params=pltpu.CompilerParams(use_tc_tiling_on_sc=False),
    )
    return f(h1, h2, src3, dst3, ew3, init1, init2)


def _leaky(x):
    return jnp.where(x > 0, x, 0.2 * x)


def _gamma_body(xb_ref, xs_ref, xsum_ref, g0t, g1t, g2t, g3t, g4w, out_ref):
    a = jnp.dot(xb_ref[...], g0t[...], preferred_element_type=jnp.float32)
    b = jnp.dot(xs_ref[...], g1t[...], preferred_element_type=jnp.float32)
    c = jnp.dot(xsum_ref[...], g2t[...], preferred_element_type=jnp.float32)
    xc = jnp.concatenate([a, b, c], axis=-1)
    t = _leaky(jnp.dot(xc, g3t[...], preferred_element_type=jnp.float32))
    out_ref[...] = jnp.dot(t, g4w[...], preferred_element_type=jnp.float32)


def _gamma_stage(xb, x_s_rep, x_sum_rep, p):
    g0t = p['gamma0'].T
    g1t = p['gamma1'].T
    g2t = p['gamma2'].T
    g3t = p['gamma3'].T
    g4w = p['gamma4_W'].T
    row = lambda i: (i, 0)
    full = lambda shape: pl.BlockSpec(shape, lambda i: (0, 0))
    out = pl.pallas_call(
        _gamma_body,
        grid=(N // _BLK,),
        in_specs=[
            pl.BlockSpec((_BLK, D), row),
            pl.BlockSpec((_BLK, D), row),
            pl.BlockSpec((_BLK, D), row),
            full((D, D)), full((D, D)), full((D, D)),
            full((3 * D, 3 * D // 2)), full((3 * D // 2, 1)),
        ],
        out_specs=pl.BlockSpec((_BLK, 1), row),
        out_shape=jax.ShapeDtypeStruct((N, 1), jnp.float32),
    )(xb, x_s_rep, x_sum_rep, g0t, g1t, g2t, g3t, g4w)
    return out[:, 0] + p['gamma4_b'][0]


def _mha(x, params, pre):
    n, L, d = x.shape
    dh = d // H
    q = (x @ params[pre + '_wq'].T).reshape(n, L, H, dh).transpose(0, 2, 1, 3).reshape(n * H, L, dh)
    kk = (x @ params[pre + '_wk'].T).reshape(n, L, H, dh).transpose(0, 2, 1, 3).reshape(n * H, L, dh)
    v = (x @ params[pre + '_wv'].T).reshape(n, L, H, dh).transpose(0, 2, 1, 3).reshape(n * H, L, dh)
    scores = jnp.tanh(q[:, :, None, :] + kk[:, None, :, :]) @ params[pre + '_ws'].T
    aw = jax.nn.softmax(scores, axis=-1)[..., 0]
    nv = jnp.einsum('bij,bjd->bid', aw, v)
    out = nv.reshape(n, H, L, dh).transpose(0, 2, 1, 3).reshape(n, L, d)
    return out @ params[pre + '_wo'].T


def kernel(x, edge_index, edge_weight, batch, states, params):
    p = params
    n = x.shape[0]
    src3 = edge_index[0].reshape(_NW, _NCH, _C)
    dst3 = edge_index[1].reshape(_NW, _NCH, _C)
    ew3 = edge_weight.reshape(_NW, _NCH, _C)
    zeros2 = jnp.zeros((_NC, _NP, D), jnp.float32)
    padrows = jnp.zeros((_NP - N, D), jnp.float32)
    pad2 = lambda h: jnp.concatenate([h, padrows])[None]
    x1 = x[:, D:]
    x2 = x[:, :D]
    for t in range(T):
        W = p['enc_cW%d' % t]
        b = p['enc_cb%d' % t]
        h1 = x1 @ W.T + b
        h2 = x2 @ W.T + b
        if t == 0:
            init1 = init2 = zeros2
        else:
            init1 = jnp.concatenate([pad2(h1), zeros2[:1]], axis=0)
            init2 = jnp.concatenate([pad2(h2), zeros2[:1]], axis=0)
        o1, o2 = _spmm2(h1, h2, src3, dst3, ew3, init1, init2)
        x1 = _leaky(o1[0, :N] + o1[1, :N])
        x2 = _leaky(o2[0, :N] + o2[1, :N])
    y = jnp.concatenate([x1 @ p['enc_w1'].T, x2 @ p['enc_w2'].T], axis=-1) @ p['enc_fc1_W'].T + p['enc_fc1_b']
    x1 = y[:, :D]
    x2 = y[:, D:]
    x1_list = []
    x2_list = []
    for t in range(T):
        g1 = _leaky((x1 + states[:, None] * p['b1_%d_a0' % t][0, 0]) @ p['b1_%d_a1W' % t].T + p['b1_%d_a1b' % t])
        g2 = _leaky((x2 + states[:, None] * p['b2_%d_a0' % t][0, 0]) @ p['b2_%d_a1W' % t].T + p['b2_%d_a1b' % t])
        h1 = g1 @ p['b1_%d_cW' % t].T + p['b1_%d_cb' % t]
        h2 = g2 @ p['b2_%d_cW' % t].T + p['b2_%d_cb' % t]
        o1, o2 = _spmm2(h1, h2, src3, dst3, ew3, zeros2, zeros2)
        x1 = _leaky(o1[0, :N] + o1[1, :N])
        x2 = _leaky(o2[0, :N] + o2[1, :N])
        x1_list.append(x1)
        x2_list.append(x2)
    x1s = jnp.stack(x1_list, axis=1)
    x2s = jnp.stack(x2_list, axis=1)
    x1_a = _mha(x1s, p, 'att1')
    x2_a = _mha(x2s, p, 'att2')
    aw1 = jax.nn.softmax((x1_a @ p['fc1_W'].T + p['fc1_b']).transpose(0, 2, 1), axis=-1)
    aw2 = jax.nn.softmax((x2_a @ p['fc2_W'].T + p['fc2_b']).transpose(0, 2, 1), axis=-1)
    x1f = jnp.einsum('bij,bjd->bid', aw1, x1_a)[:, 0, :] + x1s.sum(axis=1)
    x2f = jnp.einsum('bij,bjd->bid', aw2, x2_a)[:, 0, :] + x2s.sum(axis=1)
    xb = _leaky(jnp.concatenate([x1f @ p['beta0'].T, x2f @ p['beta1'].T], axis=-1) @ p['beta2_W'].T + p['beta2_b'])
    sel = states == 1.0
    batch_num = jax.ops.segment_sum(jnp.ones((n,), dtype=jnp.int32), batch, num_segments=NG)
    x_s = jax.ops.segment_sum(jnp.where(sel[:, None], xb, 0.0), batch, num_segments=NG)
    x_sum = jax.ops.segment_sum(xb, batch, num_segments=NG)
    x_s_rep = jnp.repeat(x_s, batch_num, axis=0, total_repeat_length=n)
    x_sum_rep = jnp.repeat(x_sum, batch_num, axis=0, total_repeat_length=n)
    return _gamma_stage(xb, x_s_rep, x_sum_rep, p)


# speedup vs baseline: 8.9917x; 1.5627x over previous
import functools

import jax
import jax.numpy as jnp
from jax import lax
from jax.experimental import pallas as pl
from jax.experimental.pallas import tpu as pltpu
from jax.experimental.pallas import tpu_sc as plsc

N = 10000
E = 320000
D = 64
T = 3
H = 4
NG = 16

_BLK = 2000

_NC = 2
_NS = 16
_NW = _NC * _NS
_EPW = E // _NW
_C = 80
_NCH = _EPW // _C
_NP = 10240
_STRIPE = _NP // _NS


_NB = 5


def _spmm2_body(h1_hbm, h2_hbm, src_hbm, dst_hbm, ew_hbm, init1_hbm, init2_hbm,
                out1_hbm, out2_hbm, si_v, di_v, ew_v, rows_v, acc_sh,
                *sems):
    gsems = sems[:_NB]
    ssems = sems[_NB:]
    c = lax.axis_index("c")
    s = lax.axis_index("s")
    wid = c * _NS + s
    stripe = pl.ds(s * _STRIPE, _STRIPE)
    pltpu.sync_copy(src_hbm.at[wid], si_v)
    pltpu.sync_copy(dst_hbm.at[wid], di_v)
    pltpu.sync_copy(ew_hbm.at[wid], ew_v)
    pltpu.sync_copy(init1_hbm.at[c, stripe], acc_sh.at[stripe])
    plsc.subcore_barrier()

    def do_dir(h_hbm, acc, g_v, sc_v):
        def gstart(i, b):
            pltpu.async_copy(h_hbm.at[g_v.at[i]], rows_v.at[b], gsems[b])

        def gwait(i, b):
            pltpu.make_async_copy(h_hbm.at[g_v.at[i]], rows_v.at[b], gsems[b]).wait()

        def sstart(i, b):
            pltpu.async_copy(rows_v.at[b], acc.at[sc_v.at[i]], ssems[b], add=True)

        def swait(i, b):
            pltpu.make_async_copy(rows_v.at[b], acc.at[sc_v.at[i]], ssems[b]).wait()

        def scale(i, b):
            def row16(e16, carry):
                wv = ew_v[i, pl.ds(e16 * 16, 16)]
                for k in range(16):
                    e = e16 * 16 + k
                    w = wv[k]
                    for j in range(D // 16):
                        rows_v[b, e, pl.ds(j * 16, 16)] = rows_v[b, e, pl.ds(j * 16, 16)] * w
                return carry

            lax.fori_loop(0, _C // 16, row16, 0)

        def slot(i, b, refill):
            gwait(i, b)
            scale(i, b)
            sstart(i, b)
            pb = (b - 1) % _NB
            if refill:
                swait(i - 1, pb)
                gstart(i - 1 + _NB, pb)

        for b in range(_NB):
            gstart(b, b)
        for b in range(_NB):
            slot(b, b, refill=(b >= 1))

        @pl.loop(1, _NCH // _NB - 1)
        def middle(g):
            i0 = g * _NB
            for b in range(_NB):
                slot(i0 + b, b, refill=True)

        i0 = _NCH - _NB
        for b in range(_NB):
            slot(i0 + b, b, refill=(b == 0))
            if b >= 1:
                swait(i0 + b - 1, (b - 1) % _NB)
        swait(_NCH - 1, (_NCH - 1) % _NB)

    do_dir(h1_hbm, acc_sh, di_v, si_v)
    plsc.subcore_barrier()
    pltpu.sync_copy(acc_sh.at[stripe], out1_hbm.at[c, stripe])
    pltpu.sync_copy(init2_hbm.at[c, stripe], acc_sh.at[stripe])
    plsc.subcore_barrier()
    do_dir(h2_hbm, acc_sh, si_v, di_v)
    plsc.subcore_barrier()
    pltpu.sync_copy(acc_sh.at[stripe], out2_hbm.at[c, stripe])


@jax.jit
def _spmm2(h1, h2, src3, dst3, ew3, init1, init2):
    mesh = plsc.VectorSubcoreMesh(core_axis_name="c", subcore_axis_name="s")
    f = pl.kernel(
        _spmm2_body,
        out_type=(jax.ShapeDtypeStruct((_NC, _NP, D), jnp.float32),
                  jax.ShapeDtypeStruct((_NC, _NP, D), jnp.float32)),
        mesh=mesh,
        scratch_types=[
            pltpu.VMEM((_NCH, _C), jnp.int32),
            pltpu.VMEM((_NCH, _C), jnp.int32),
            pltpu.VMEM((_NCH, _C), jnp.float32),
            pltpu.VMEM((_NB, _C, D), jnp.float32),
            pltpu.VMEM_SHARED((_NP, D), jnp.float32),
        ] + [pltpu.SemaphoreType.DMA] * (2 * _NB),
        compiler_params=pltpu.CompilerParams(use_tc_tiling_on_sc=False),
    )
    return f(h1, h2, src3, dst3, ew3, init1, init2)


def _leaky(x):
    return jnp.where(x > 0, x, 0.2 * x)


_ROW = lambda i: (i, 0)
_FULL = lambda shape: pl.BlockSpec(shape, lambda i: (0, 0))


def _rowspec(width):
    return pl.BlockSpec((_BLK, width), _ROW)


def _dot(a, b):
    return jnp.dot(a, b, preferred_element_type=jnp.float32)


def _pre0_body(x_ref, w0t, b0, h1_ref, h2_ref):
    x = x_ref[...]
    h1_ref[...] = _dot(x[:, D:], w0t[...]) + b0[...]
    h2_ref[...] = _dot(x[:, :D], w0t[...]) + b0[...]


def _pre0(x, w0t, b0):
    return pl.pallas_call(
        _pre0_body,
        grid=(N // _BLK,),
        in_specs=[_rowspec(2 * D), _FULL((D, D)), _FULL((1, D))],
        out_specs=(_rowspec(D), _rowspec(D)),
        out_shape=(jax.ShapeDtypeStruct((N, D), jnp.float32),
                   jax.ShapeDtypeStruct((N, D), jnp.float32)),
    )(x, w0t, b0)


def _gapenc_body(o1a, o1b, o2a, o2b, wt, b, h1_ref, h2_ref):
    x1 = _leaky(o1a[...] + o1b[...])
    x2 = _leaky(o2a[...] + o2b[...])
    h1_ref[...] = _dot(x1, wt[...]) + b[...]
    h2_ref[...] = _dot(x2, wt[...]) + b[...]


def _gapenc(o1a, o1b, o2a, o2b, wt, b):
    return pl.pallas_call(
        _gapenc_body,
        grid=(N // _BLK,),
        in_specs=[_rowspec(D)] * 4 + [_FULL((D, D)), _FULL((1, D))],
        out_specs=(_rowspec(D), _rowspec(D)),
        out_shape=(jax.ShapeDtypeStruct((N, D), jnp.float32),
                   jax.ShapeDtypeStruct((N, D), jnp.float32)),
    )(o1a, o1b, o2a, o2b, wt, b)


def _branch_pre(x, st, a0, a1wt, a1b, cwt, cb):
    g = _leaky(_dot(x + st * a0[...], a1wt[...]) + a1b[...])
    return _dot(g, cwt[...]) + cb[...]


def _gap3_body(o1a, o1b, o2a, o2b, st_ref, w1t, w2t, fc1wt, fc1b,
               a0_1, a1wt_1, a1b_1, cwt_1, cb_1,
               a0_2, a1wt_2, a1b_2, cwt_2, cb_2,
               x1_ref, x2_ref, h1_ref, h2_ref):
    x1 = _leaky(o1a[...] + o1b[...])
    x2 = _leaky(o2a[...] + o2b[...])
    xc = jnp.concatenate([_dot(x1, w1t[...]), _dot(x2, w2t[...])], axis=-1)
    y = _dot(xc, fc1wt[...]) + fc1b[...]
    x1n = y[:, :D]
    x2n = y[:, D:]
    st = st_ref[...]
    x1_ref[...] = x1n
    x2_ref[...] = x2n
    h1_ref[...] = _branch_pre(x1n, st, a0_1, a1wt_1, a1b_1, cwt_1, cb_1)
    h2_ref[...] = _branch_pre(x2n, st, a0_2, a1wt_2, a1b_2, cwt_2, cb_2)


def _gap3(o1a, o1b, o2a, o2b, st, p):
    return pl.pallas_call(
        _gap3_body,
        grid=(N // _BLK,),
        in_specs=[_rowspec(D)] * 4 + [_rowspec(1)]
        + [_FULL((D, D)), _FULL((D, D)), _FULL((2 * D, 2 * D)), _FULL((1, 2 * D))]
        + [_FULL((1, 1)), _FULL((D, D)), _FULL((1, D)), _FULL((D, D)), _FULL((1, D))] * 2,
        out_specs=(_rowspec(D),) * 4,
        out_shape=(jax.ShapeDtypeStruct((N, D), jnp.float32),) * 4,
    )(o1a, o1b, o2a, o2b, st,
      p['enc_w1'].T, p['enc_w2'].T, p['enc_fc1_W'].T, p['enc_fc1_b'][None],
      p['b1_0_a0'], p['b1_0_a1W'].T, p['b1_0_a1b'][None], p['b1_0_cW'].T, p['b1_0_cb'][None],
      p['b2_0_a0'], p['b2_0_a1W'].T, p['b2_0_a1b'][None], p['b2_0_cW'].T, p['b2_0_cb'][None])


def _gapbr_body(o1a, o1b, o2a, o2b, st_ref,
                a0_1, a1wt_1, a1b_1, cwt_1, cb_1,
                a0_2, a1wt_2, a1b_2, cwt_2, cb_2,
                x1_ref, x2_ref, h1_ref, h2_ref):
    x1 = _leaky(o1a[...] + o1b[...])
    x2 = _leaky(o2a[...] + o2b[...])
    st = st_ref[...]
    x1_ref[...] = x1
    x2_ref[...] = x2
    h1_ref[...] = _branch_pre(x1, st, a0_1, a1wt_1, a1b_1, cwt_1, cb_1)
    h2_ref[...] = _branch_pre(x2, st, a0_2, a1wt_2, a1b_2, cwt_2, cb_2)


def _gapbr(o1a, o1b, o2a, o2b, st, p, t):
    return pl.pallas_call(
        _gapbr_body,
        grid=(N // _BLK,),
        in_specs=[_rowspec(D)] * 4 + [_rowspec(1)]
        + [_FULL((1, 1)), _FULL((D, D)), _FULL((1, D)), _FULL((D, D)), _FULL((1, D))] * 2,
        out_specs=(_rowspec(D),) * 4,
        out_shape=(jax.ShapeDtypeStruct((N, D), jnp.float32),) * 4,
    )(o1a, o1b, o2a, o2b, st,
      p['b1_%d_a0' % t], p['b1_%d_a1W' % t].T, p['b1_%d_a1b' % t][None],
      p['b1_%d_cW' % t].T, p['b1_%d_cb' % t][None],
      p['b2_%d_a0' % t], p['b2_%d_a1W' % t].T, p['b2_%d_a1b' % t][None],
      p['b2_%d_cW' % t].T, p['b2_%d_cb' % t][None])


def _att_body(o1a, o1b, o2a, o2b, x1_0, x1_1, x2_0, x2_1,
              m1, m2, b0t, b1t, b2wt, b2b, xb_ref):
    s1 = x1_0[...] + x1_1[...] + _leaky(o1a[...] + o1b[...])
    s2 = x2_0[...] + x2_1[...] + _leaky(o2a[...] + o2b[...])
    x1f = _dot(s1, m1[...]) + s1
    x2f = _dot(s2, m2[...]) + s2
    xc = jnp.concatenate([_dot(x1f, b0t[...]), _dot(x2f, b1t[...])], axis=-1)
    xb_ref[...] = _leaky(_dot(xc, b2wt[...]) + b2b[...])


def _att_final(o1a, o1b, o2a, o2b, x1_0, x1_1, x2_0, x2_1, p):
    m1 = p['att1_wv'].T @ p['att1_wo'].T
    m2 = p['att2_wv'].T @ p['att2_wo'].T
    return pl.pallas_call(
        _att_body,
        grid=(N // _BLK,),
        in_specs=[_rowspec(D)] * 8
        + [_FULL((D, D)), _FULL((D, D)),
           _FULL((D, D)), _FULL((D, D)), _FULL((2 * D, D)), _FULL((1, D))],
        out_specs=_rowspec(D),
        out_shape=jax.ShapeDtypeStruct((N, D), jnp.float32),
    )(o1a, o1b, o2a, o2b, x1_0, x1_1, x2_0, x2_1,
      m1, m2, p['beta0'].T, p['beta1'].T, p['beta2_W'].T, p['beta2_b'][None])


def _seg_body(xb_ref, st_ref, bt_ref, xs_ref, xsum_ref):
    g = pl.program_id(0)
    onehot = (bt_ref[...] == lax.broadcasted_iota(jnp.int32, (_BLK, NG), 1)
              ).astype(jnp.float32)
    xb = xb_ref[...]
    contract = (((0,), (0,)), ((), ()))
    xsum_p = lax.dot_general(onehot, xb, contract,
                             preferred_element_type=jnp.float32)
    xs_p = lax.dot_general(onehot, xb * st_ref[...], contract,
                           preferred_element_type=jnp.float32)

    @pl.when(g == 0)
    def _():
        xs_ref[...] = jnp.zeros_like(xs_ref)
        xsum_ref[...] = jnp.zeros_like(xsum_ref)

    xs_ref[...] += xs_p
    xsum_ref[...] += xsum_p


def _seg(xb, st, bt):
    return pl.pallas_call(
        _seg_body,
        grid=(N // _BLK,),
        in_specs=[_rowspec(D), _rowspec(1), pl.BlockSpec((_BLK, 1), _ROW)],
        out_specs=(_FULL((NG, D)), _FULL((NG, D))),
        out_shape=(jax.ShapeDtypeStruct((NG, D), jnp.float32),
                   jax.ShapeDtypeStruct((NG, D), jnp.float32)),
    )(xb, st, bt)


def _gamma_body(xb_ref, bt_ref, xs_ref, xsum_ref, g0t, g1t, g2t, g3t, g4w, out_ref):
    onehot = (bt_ref[...] == lax.broadcasted_iota(jnp.int32, (_BLK, NG), 1)
              ).astype(jnp.float32)
    xs_rep = _dot(onehot, xs_ref[...])
    xsum_rep = _dot(onehot, xsum_ref[...])
    xc = jnp.concatenate([_dot(xb_ref[...], g0t[...]),
                          _dot(xs_rep, g1t[...]),
                          _dot(xsum_rep, g2t[...])], axis=-1)
    t = _leaky(_dot(xc, g3t[...]))
    out_ref[...] = _dot(t, g4w[...])


def _gamma_stage(xb, bt, x_s, x_sum, p):
    out = pl.pallas_call(
        _gamma_body,
        grid=(N // _BLK,),
        in_specs=[_rowspec(D), pl.BlockSpec((_BLK, 1), _ROW),
                  _FULL((NG, D)), _FULL((NG, D)),
                  _FULL((D, D)), _FULL((D, D)), _FULL((D, D)),
                  _FULL((3 * D, 3 * D // 2)), _FULL((3 * D // 2, 1))],
        out_specs=_rowspec(1),
        out_shape=jax.ShapeDtypeStruct((N, 1), jnp.float32),
    )(xb, bt, x_s, x_sum,
      p['gamma0'].T, p['gamma1'].T, p['gamma2'].T, p['gamma3'].T, p['gamma4_W'].T)
    return out[:, 0] + p['gamma4_b'][0]


def kernel(x, edge_index, edge_weight, batch, states, params):
    p = params
    src3 = edge_index[0].reshape(_NW, _NCH, _C)
    dst3 = edge_index[1].reshape(_NW, _NCH, _C)
    ew3 = edge_weight.reshape(_NW, _NCH, _C)
    zeros2 = jnp.zeros((_NC, _NP, D), jnp.float32)
    padrows = jnp.zeros((_NP - N, D), jnp.float32)
    pad2 = lambda h: jnp.concatenate([h, padrows])[None]
    st = states[:, None]
    bt = batch[:, None]

    h1, h2 = _pre0(x, p['enc_cW0'].T, p['enc_cb0'][None])
    o1, o2 = _spmm2(h1, h2, src3, dst3, ew3, zeros2, zeros2)
    for t in (1, 2):
        h1, h2 = _gapenc(o1[0], o1[1], o2[0], o2[1],
                         p['enc_cW%d' % t].T, p['enc_cb%d' % t][None])
        init1 = jnp.concatenate([pad2(h1), zeros2[:1]], axis=0)
        init2 = jnp.concatenate([pad2(h2), zeros2[:1]], axis=0)
        o1, o2 = _spmm2(h1, h2, src3, dst3, ew3, init1, init2)

    x1c, x2c, h1, h2 = _gap3(o1[0], o1[1], o2[0], o2[1], st, p)
    o1, o2 = _spmm2(h1, h2, src3, dst3, ew3, zeros2, zeros2)
    x1_0o, x2_0o, h1, h2 = _gapbr(o1[0], o1[1], o2[0], o2[1], st, p, 1)
    o1, o2 = _spmm2(h1, h2, src3, dst3, ew3, zeros2, zeros2)
    x1_1o, x2_1o, h1, h2 = _gapbr(o1[0], o1[1], o2[0], o2[1], st, p, 2)
    o1, o2 = _spmm2(h1, h2, src3, dst3, ew3, zeros2, zeros2)

    xb = _att_final(o1[0], o1[1], o2[0], o2[1], x1_0o, x1_1o, x2_0o, x2_1o, p)
    x_s, x_sum = _seg(xb, st, bt)
    return _gamma_stage(xb, bt, x_s, x_sum, p)
